# baseline (speedup 1.0000x reference)
import functools

import jax
import jax.numpy as jnp
from jax import lax
from jax.experimental import pallas as pl
from jax.experimental.pallas import tpu as pltpu
from jax.experimental.pallas import tpu_sc as plsc

B, L = 4096, 200
E, H, C = 64, 128, 10
NC, NS, LANES = 2, 16, 16
NW = NC * NS
BPW = B // NW
NCHUNK = 2
LCS = (104, 96)
LOFF = (0, 104)
LCMAX = 104
PCHUNKS = E // 32
NITEMS = 2 * NCHUNK
NSLOT = 4
LOOKAHEAD = 2
NPAIR = BPW // 2


def _pool_body(idx_h, tab_h, out_h, idx_v, rows_v, out_v, sem0, sem1, sem2, sem3):
    wid = lax.axis_index("s") * NC + lax.axis_index("c")
    base = wid * BPW
    sems = (sem0, sem1, sem2, sem3)
    mask_hi = jnp.full((LANES,), -65536, jnp.int32)
    evens = lax.iota(jnp.int32, LANES) * 2

    pltpu.sync_copy(idx_h.at[pl.ds(base, BPW)], idx_v)

    def copy_for(p, k):
        db, j = divmod(k, 2)
        return pltpu.make_async_copy(
            tab_h.at[idx_v.at[p * 2 + db, pl.ds(LOFF[j], LCS[j])]],
            rows_v.at[k % NSLOT, pl.ds(0, LCS[j])], sems[k % NSLOT])

    for k in range(LOOKAHEAD):
        copy_for(0, k).start()

    def per_pair(p, carry):
        accs = None
        for k in range(NITEMS):
            db, j = divmod(k, 2)
            ka = k + LOOKAHEAD
            if ka < NITEMS:
                copy_for(p, ka).start()
            else:
                @pl.when(p < NPAIR - 1)
                def _():
                    copy_for(p + 1, ka - NITEMS).start()
            copy_for(p, k).wait()

            if j == 0:
                accs = tuple(
                    jnp.zeros((LANES,), jnp.float32)
                    for _ in range(2 * PCHUNKS))

            @plsc.parallel_loop(0, LCS[j], unroll=4, carry=accs)
            def accs(r, accs, _slot=k % NSLOT):
                new = []
                for c in range(PCHUNKS):
                    packed = plsc.bitcast(
                        rows_v[_slot, r, pl.ds(32 * c, 32)], jnp.int32)
                    lo = plsc.bitcast(packed << 16, jnp.float32)
                    hi = plsc.bitcast(packed & mask_hi, jnp.float32)
                    new.append(accs[2 * c] + lo)
                    new.append(accs[2 * c + 1] + hi)
                return tuple(new)

            if j == 1:
                dst = out_v.at[p * 2 + db]
                for c in range(PCHUNKS):
                    plsc.store_scatter(
                        dst, [evens + 32 * c], accs[2 * c] * (1.0 / L))
                    plsc.store_scatter(
                        dst, [evens + (32 * c + 1)], accs[2 * c + 1] * (1.0 / L))
        return carry

    lax.fori_loop(0, NPAIR, per_pair, 0)
    pltpu.sync_copy(out_v, out_h.at[pl.ds(base, BPW)])


_pool = pl.kernel(
    _pool_body,
    out_type=jax.ShapeDtypeStruct((B, E), jnp.float32),
    mesh=plsc.VectorSubcoreMesh(
        core_axis_name="c", subcore_axis_name="s",
        num_cores=NC, num_subcores=NS,
    ),
    scratch_types=[
        pltpu.VMEM((BPW, L), jnp.int32),
        pltpu.VMEM((NSLOT, LCMAX, E), jnp.bfloat16),
        pltpu.VMEM((BPW, E), jnp.float32),
        pltpu.SemaphoreType.DMA,
        pltpu.SemaphoreType.DMA,
        pltpu.SemaphoreType.DMA,
        pltpu.SemaphoreType.DMA,
    ],
    compiler_params=pltpu.CompilerParams(
        use_tc_tiling_on_sc=False, needs_layout_passes=False),
)


def _mlp_body(x1_ref, x2_ref, x3_ref, w1_ref, b1_ref, w2_ref, b2_ref, o_ref):
    h = jnp.dot(x1_ref[...], w1_ref[0:E], preferred_element_type=jnp.float32)
    h += jnp.dot(x2_ref[...], w1_ref[E:2 * E], preferred_element_type=jnp.float32)
    h += jnp.dot(x3_ref[...], w1_ref[2 * E:3 * E], preferred_element_type=jnp.float32)
    h = jnp.maximum(h + b1_ref[...], 0.0)
    o_ref[...] = jnp.dot(h, w2_ref[...], preferred_element_type=jnp.float32) + b2_ref[...]


_mlp = pl.pallas_call(
    _mlp_body,
    out_shape=jax.ShapeDtypeStruct((B, C), jnp.float32),
)


@jax.jit
def kernel(bos, bigram, trigram, uni_table, bi_table, tri_table,
           fc1_w, fc1_b, fc2_w, fc2_b):
    uni_bf = uni_table.astype(jnp.bfloat16)
    bi_bf = bi_table.astype(jnp.bfloat16)
    tri_bf = tri_table.astype(jnp.bfloat16)
    x1 = _pool(bos, uni_bf)
    x2 = _pool(bigram, bi_bf)
    x3 = _pool(trigram, tri_bf)
    return _mlp(x1, x2, x3, fc1_w, fc1_b.reshape(1, H), fc2_w, fc2_b.reshape(1, C))


# speedup vs baseline: 1.3424x; 1.3424x over previous
import functools

import jax
import jax.numpy as jnp
from jax import lax
from jax.experimental import pallas as pl
from jax.experimental.pallas import tpu as pltpu
from jax.experimental.pallas import tpu_sc as plsc

B, L = 4096, 200
E, H, C = 64, 128, 10
NC, NS, LANES = 2, 16, 16
NW = NC * NS
BPW = B // NW
NCHUNK = 2
LCS = (104, 96)
LOFF = (0, 104)
LCMAX = 104
ECHUNKS = E // LANES
NITEMS = 2 * NCHUNK
NSLOT = 4
LOOKAHEAD = 2
NPAIR = BPW // 2


def _pool_body(idx_h, tab_h, out_h, idx_v, rows_v, out_v, sem0, sem1, sem2, sem3):
    wid = lax.axis_index("s") * NC + lax.axis_index("c")
    base = wid * BPW
    sems = (sem0, sem1, sem2, sem3)

    pltpu.sync_copy(idx_h.at[pl.ds(base, BPW)], idx_v)

    def copy_for(p, k):
        db, j = divmod(k, 2)
        return pltpu.make_async_copy(
            tab_h.at[idx_v.at[p * 2 + db, pl.ds(LOFF[j], LCS[j])]],
            rows_v.at[k % NSLOT, pl.ds(0, LCS[j])], sems[k % NSLOT])

    for k in range(LOOKAHEAD):
        copy_for(0, k).start()

    def per_pair(p, carry):
        accs = None
        for k in range(NITEMS):
            db, j = divmod(k, 2)
            ka = k + LOOKAHEAD
            if ka < NITEMS:
                copy_for(p, ka).start()
            else:
                @pl.when(p < NPAIR - 1)
                def _():
                    copy_for(p + 1, ka - NITEMS).start()
            copy_for(p, k).wait()

            if j == 0:
                accs = tuple(
                    jnp.zeros((LANES,), jnp.float32) for _ in range(ECHUNKS))

            @plsc.parallel_loop(0, LCS[j], unroll=4, carry=accs)
            def accs(r, accs, _slot=k % NSLOT):
                return tuple(
                    accs[c] + rows_v[_slot, r, pl.ds(LANES * c, LANES)]
                    for c in range(ECHUNKS)
                )

            if j == 1:
                for c in range(ECHUNKS):
                    out_v[p * 2 + db, pl.ds(c * LANES, LANES)] = (
                        accs[c] * (1.0 / L))
        return carry

    lax.fori_loop(0, NPAIR, per_pair, 0)
    pltpu.sync_copy(out_v, out_h.at[pl.ds(base, BPW)])


_pool = pl.kernel(
    _pool_body,
    out_type=jax.ShapeDtypeStruct((B, E), jnp.float32),
    mesh=plsc.VectorSubcoreMesh(
        core_axis_name="c", subcore_axis_name="s",
        num_cores=NC, num_subcores=NS,
    ),
    scratch_types=[
        pltpu.VMEM((BPW, L), jnp.int32),
        pltpu.VMEM((NSLOT, LCMAX, E), jnp.float32),
        pltpu.VMEM((BPW, E), jnp.float32),
        pltpu.SemaphoreType.DMA,
        pltpu.SemaphoreType.DMA,
        pltpu.SemaphoreType.DMA,
        pltpu.SemaphoreType.DMA,
    ],
    compiler_params=pltpu.CompilerParams(use_tc_tiling_on_sc=False),
)


def _mlp_body(x1_ref, x2_ref, x3_ref, w1_ref, b1_ref, w2_ref, b2_ref, o_ref):
    h = jnp.dot(x1_ref[...], w1_ref[0:E], preferred_element_type=jnp.float32)
    h += jnp.dot(x2_ref[...], w1_ref[E:2 * E], preferred_element_type=jnp.float32)
    h += jnp.dot(x3_ref[...], w1_ref[2 * E:3 * E], preferred_element_type=jnp.float32)
    h = jnp.maximum(h + b1_ref[...], 0.0)
    o_ref[...] = jnp.dot(h, w2_ref[...], preferred_element_type=jnp.float32) + b2_ref[...]


_mlp = pl.pallas_call(
    _mlp_body,
    out_shape=jax.ShapeDtypeStruct((B, C), jnp.float32),
)


@jax.jit
def kernel(bos, bigram, trigram, uni_table, bi_table, tri_table,
           fc1_w, fc1_b, fc2_w, fc2_b):
    x1 = _pool(bos, uni_table)
    x2 = _pool(bigram, bi_table)
    x3 = _pool(trigram, tri_table)
    return _mlp(x1, x2, x3, fc1_w, fc1_b.reshape(1, H), fc2_w, fc2_b.reshape(1, C))
